# baseline (speedup 1.0000x reference)
import functools

import jax
import jax.numpy as jnp
from jax import lax
from jax.experimental import pallas as pl
from jax.experimental.pallas import tpu as pltpu
from jax.experimental.pallas import tpu_sc as plsc

EMB_D = 32
NUM_CORES = 2
NUM_SUBCORES = 16
NW = NUM_CORES * NUM_SUBCORES
CHUNK = 128
SPITCH = EMB_D + 1


@functools.lru_cache(maxsize=None)
def _make_lookup(NT: int, NB_TOTAL: int, V: int):
    NB = NB_TOTAL // NW
    n_g = NB // CHUNK
    assert NT % 2 == 0 and NB % CHUNK == 0
    n_iter = NT // 2
    mesh = plsc.VectorSubcoreMesh(core_axis_name="c", subcore_axis_name="s")

    @functools.partial(
        pl.kernel,
        mesh=mesh,
        out_type=jax.ShapeDtypeStruct((NT, EMB_D, NB_TOTAL), jnp.float32),
        scratch_types=[
            pltpu.VMEM((NT, NB), jnp.int32),
            pltpu.VMEM((NB, EMB_D), jnp.float32),
            pltpu.VMEM((NB, EMB_D), jnp.float32),
            pltpu.VMEM((NB * SPITCH,), jnp.float32),
            pltpu.VMEM((EMB_D, NB), jnp.float32),
            pltpu.VMEM((EMB_D, NB), jnp.float32),
            pltpu.SemaphoreType.DMA,
            pltpu.SemaphoreType.DMA,
            pltpu.SemaphoreType.DMA,
        ],
        compiler_params=pltpu.CompilerParams(
            use_tc_tiling_on_sc=False, needs_layout_passes=False
        ),
    )
    def lookup_kernel(xt_hbm, table_hbm, out_hbm, idx_v, src_a, src_b,
                      spad, dst_a, dst_b, gsem, wsem_a, wsem_b):
        wid = lax.axis_index("s") * NUM_CORES + lax.axis_index("c")
        n0 = wid * NB
        lane = lax.iota(jnp.int32, 16)

        pltpu.sync_copy(xt_hbm.at[pl.ds(0, NT), pl.ds(n0, NB)], idx_v)

        def fire_gathers(t, sb):
            for b in range(n_g):
                pltpu.async_copy(
                    table_hbm.at[idx_v.at[t, pl.ds(b * CHUNK, CHUNK)]],
                    sb.at[pl.ds(b * CHUNK, CHUNK)],
                    gsem,
                )

        def drain_gathers(sb):
            pltpu.make_async_copy(table_hbm.at[pl.ds(0, NB)], sb, gsem).wait()

        def transpose(sb, db):
            def row(n, carry):
                p = n * SPITCH
                spad[pl.ds(p, 16)] = sb[n, pl.ds(0, 16)]
                spad[pl.ds(p + 16, 16)] = sb[n, pl.ds(16, 16)]
                return carry

            lax.fori_loop(0, NB, row, 0, unroll=16)

            def blk(nb, carry):
                flat = (nb * 16 + lane) * SPITCH
                base = nb * 16
                for j in range(EMB_D):
                    v = plsc.load_gather(spad, [flat + j])
                    db[j, pl.ds(base, 16)] = v
                return carry

            lax.fori_loop(0, NB // 16, blk, 0, unroll=4)

        def fire_write(t, db, ws):
            pltpu.async_copy(
                db,
                out_hbm.at[t, pl.ds(0, EMB_D), pl.ds(n0, NB)],
                ws,
            )

        def drain_write(db, ws):
            pltpu.make_async_copy(
                out_hbm.at[0, pl.ds(0, EMB_D), pl.ds(0, NB)], db, ws
            ).wait()

        fire_gathers(0, src_a)

        def body(u, carry):
            t0 = 2 * u
            t1 = t0 + 1
            drain_gathers(src_a)
            fire_gathers(t1, src_b)

            @pl.when(u > 0)
            def _():
                drain_write(dst_a, wsem_a)

            transpose(src_a, dst_a)
            fire_write(t0, dst_a, wsem_a)

            drain_gathers(src_b)

            @pl.when(u < n_iter - 1)
            def _():
                fire_gathers(t0 + 2, src_a)

            @pl.when(u > 0)
            def _():
                drain_write(dst_b, wsem_b)

            transpose(src_b, dst_b)
            fire_write(t1, dst_b, wsem_b)
            return carry

        lax.fori_loop(0, n_iter, body, 0)
        drain_write(dst_a, wsem_a)
        drain_write(dst_b, wsem_b)

    return lookup_kernel


def kernel(x, W):
    rows, cols = x.shape
    xt = x.T
    out3 = _make_lookup(cols, rows, W.shape[0])(xt, W)
    return out3.transpose(2, 0, 1)
